# baseline (speedup 1.0000x reference)
import functools

import jax
import jax.numpy as jnp
from jax import lax
from jax.experimental import pallas as pl
from jax.experimental.pallas import tpu as pltpu
from jax.experimental.pallas import tpu_sc as plsc

NC = 2
NS = 16
NW = NC * NS

VOCAB = 100000
D = 1024
B_FLAT = 4 * 8192
B_PER_W = B_FLAT // NW
CHUNK = 16
NCHUNK = B_PER_W // CHUNK
NBUF = 6
W_PER_ROW = 8192 // B_PER_W
PAIR_SETS = ((0, 1), (2, 3), (4, 5))


def _gather_body(table_hbm, idx_hbm, out_hbm, idx_v, rows0, rows1, rows2,
                 rows3, rows4, rows5, gsem0, gsem1, gsem2, gsem3, gsem4,
                 gsem5, osem0, osem1, osem2, osem3, osem4, osem5):
    wid = lax.axis_index("s") * NC + lax.axis_index("c")
    base = wid * B_PER_W
    row = wid // W_PER_ROW
    col = (wid % W_PER_ROW) * B_PER_W
    pltpu.sync_copy(idx_hbm.at[row, pl.ds(col, B_PER_W)], idx_v)

    rows = (rows0, rows1, rows2, rows3, rows4, rows5)
    gsem = (gsem0, gsem1, gsem2, gsem3, gsem4, gsem5)
    osem = (osem0, osem1, osem2, osem3, osem4, osem5)

    def idx_c(i):
        return idx_v.at[pl.ds(i * CHUNK, CHUNK)]

    def out_c(i):
        return out_hbm.at[pl.ds(base + i * CHUNK, CHUNK)]

    NSET = len(PAIR_SETS)

    def pair(p, set_idx, waited):
        b0, b1 = PAIR_SETS[set_idx]
        i0, i1 = 2 * p, 2 * p + 1
        if waited:
            lag = 2 * NSET
            pltpu.make_async_copy(rows[b0], out_c(i0 - lag), osem[b0]).wait()
            pltpu.make_async_copy(rows[b1], out_c(i1 - lag), osem[b1]).wait()
        g0 = pltpu.async_copy(table_hbm.at[idx_c(i0)], rows[b0], gsem[b0])
        g1 = pltpu.async_copy(table_hbm.at[idx_c(i1)], rows[b1], gsem[b1])
        g0.wait()
        g1.wait()
        pltpu.async_copy(rows[b0], out_c(i0), osem[b0])
        pltpu.async_copy(rows[b1], out_c(i1), osem[b1])

    NPAIR = NCHUNK // 2
    for p in range(NSET):
        pair(p, p, waited=False)

    mid_hi = NSET * ((NPAIR - NSET) // NSET) + NSET

    @pl.loop(NSET, mid_hi, step=NSET)
    def _middle(pv):
        for dp in range(NSET):
            pair(pv + dp, dp, waited=True)

    for p in range(mid_hi, NPAIR):
        pair(p, p % NSET, waited=True)

    for p in range(NPAIR - NSET, NPAIR):
        b0, b1 = PAIR_SETS[p % NSET]
        pltpu.make_async_copy(rows[b0], out_c(2 * p), osem[b0]).wait()
        pltpu.make_async_copy(rows[b1], out_c(2 * p + 1), osem[b1]).wait()


@functools.partial(jax.jit, static_argnames=())
def _sc_gather(wte, ids2d):
    mesh = plsc.VectorSubcoreMesh(
        core_axis_name="c", subcore_axis_name="s",
        num_cores=NC, num_subcores=NS,
    )
    return pl.kernel(
        _gather_body,
        out_type=jax.ShapeDtypeStruct((B_FLAT, D), jnp.float32),
        mesh=mesh,
        scratch_types=(
            [pltpu.VMEM((B_PER_W,), jnp.int32)]
            + [pltpu.VMEM((CHUNK, D), jnp.float32)] * NBUF
            + [pltpu.SemaphoreType.DMA] * (2 * NBUF)
        ),
    )(wte, ids2d)


def _mask_body(am_ref, out_ref):
    m = am_ref[...].astype(jnp.bfloat16)
    out_ref[...] = (1.0 - m) * jnp.finfo(jnp.bfloat16).min


def _mask_expand(attention_mask):
    b, s = attention_mask.shape
    return pl.pallas_call(
        _mask_body,
        out_shape=jax.ShapeDtypeStruct((b, s), jnp.bfloat16),
    )(attention_mask)


def kernel(input_ids, attention_mask, wte):
    input_shape = input_ids.shape
    hidden = _sc_gather(wte, input_ids)
    hidden = hidden.reshape(*input_shape, D)
    batch = input_ids.reshape(-1, input_shape[-1]).shape[0]
    am = _mask_expand(attention_mask.reshape(batch, -1))
    am = am[:, None, None, :]
    return (hidden, am)


# speedup vs baseline: 1.0064x; 1.0064x over previous
import functools

import jax
import jax.numpy as jnp
from jax import lax
from jax.experimental import pallas as pl
from jax.experimental.pallas import tpu as pltpu
from jax.experimental.pallas import tpu_sc as plsc

NC = 2
NS = 16
NW = NC * NS

VOCAB = 100000
D = 1024
B_FLAT = 4 * 8192
B_PER_W = B_FLAT // NW
CHUNK = 16
NCHUNK = B_PER_W // CHUNK
NBUF = 4
W_PER_ROW = 8192 // B_PER_W


def _gather_body(table_hbm, idx_hbm, out_hbm, idx_v, rows0, rows1, rows2,
                 rows3, gsem0, gsem1, gsem2, gsem3, osem0, osem1, osem2,
                 osem3):
    wid = lax.axis_index("s") * NC + lax.axis_index("c")
    base = wid * B_PER_W
    row = wid // W_PER_ROW
    col = (wid % W_PER_ROW) * B_PER_W
    pltpu.sync_copy(idx_hbm.at[row, pl.ds(col, B_PER_W)], idx_v)

    rows = (rows0, rows1, rows2, rows3)
    gsem = (gsem0, gsem1, gsem2, gsem3)
    osem = (osem0, osem1, osem2, osem3)

    def idx_c(i):
        return idx_v.at[pl.ds(i * CHUNK, CHUNK)]

    def out_c(i):
        return out_hbm.at[pl.ds(base + i * CHUNK, CHUNK)]

    def pair(p, parity, waited):
        b0, b1 = (0, 1) if parity == 0 else (2, 3)
        i0, i1 = 2 * p, 2 * p + 1
        if waited:
            pltpu.make_async_copy(rows[b0], out_c(i0 - 4), osem[b0]).wait()
            pltpu.make_async_copy(rows[b1], out_c(i1 - 4), osem[b1]).wait()
        g0 = pltpu.async_copy(table_hbm.at[idx_c(i0)], rows[b0], gsem[b0])
        g1 = pltpu.async_copy(table_hbm.at[idx_c(i1)], rows[b1], gsem[b1])
        g0.wait()
        g1.wait()
        pltpu.async_copy(rows[b0], out_c(i0), osem[b0])
        pltpu.async_copy(rows[b1], out_c(i1), osem[b1])

    NPAIR = NCHUNK // 2
    pair(0, 0, waited=False)
    pair(1, 1, waited=False)

    @pl.loop(2, NPAIR, step=2)
    def _middle(pv):
        for dp in range(2):
            pair(pv + dp, dp, waited=True)

    for j in range(NBUF):
        pltpu.make_async_copy(rows[j], out_c(NCHUNK - NBUF + j), osem[j]).wait()


@functools.partial(jax.jit, static_argnames=())
def _sc_gather(wte, ids2d):
    mesh = plsc.VectorSubcoreMesh(
        core_axis_name="c", subcore_axis_name="s",
        num_cores=NC, num_subcores=NS,
    )
    return pl.kernel(
        _gather_body,
        out_type=jax.ShapeDtypeStruct((B_FLAT, D), jnp.float32),
        mesh=mesh,
        scratch_types=(
            [pltpu.VMEM((B_PER_W,), jnp.int32)]
            + [pltpu.VMEM((CHUNK, D), jnp.float32)] * NBUF
            + [pltpu.SemaphoreType.DMA] * (2 * NBUF)
        ),
    )(wte, ids2d)


def _mask_body(am_ref, out_ref):
    m = am_ref[...].astype(jnp.bfloat16)
    out_ref[...] = (1.0 - m) * jnp.finfo(jnp.bfloat16).min


def _mask_expand(attention_mask):
    b, s = attention_mask.shape
    return pl.pallas_call(
        _mask_body,
        out_shape=jax.ShapeDtypeStruct((b, s), jnp.bfloat16),
    )(attention_mask)


def kernel(input_ids, attention_mask, wte):
    input_shape = input_ids.shape
    hidden = _sc_gather(wte, input_ids)
    hidden = hidden.reshape(*input_shape, D)
    batch = input_ids.reshape(-1, input_shape[-1]).shape[0]
    am = _mask_expand(attention_mask.reshape(batch, -1))
    am = am[:, None, None, :]
    return (hidden, am)


# speedup vs baseline: 1.0103x; 1.0039x over previous
import functools

import jax
import jax.numpy as jnp
from jax import lax
from jax.experimental import pallas as pl
from jax.experimental.pallas import tpu as pltpu
from jax.experimental.pallas import tpu_sc as plsc

NC = 2
NS = 16
NW = NC * NS

VOCAB = 100000
D = 1024
B_FLAT = 4 * 8192
B_PER_W = B_FLAT // NW
CHUNK = 16
PAIR = 2 * CHUNK
NPAIR = B_PER_W // PAIR
W_PER_ROW = 8192 // B_PER_W


def _gather_body(table_hbm, idx_hbm, out_hbm, idx_v, rows_v,
                 gsem0, gsem1, osem0, osem1):
    wid = lax.axis_index("s") * NC + lax.axis_index("c")
    base = wid * B_PER_W
    row = wid // W_PER_ROW
    col = (wid % W_PER_ROW) * B_PER_W
    pltpu.sync_copy(idx_hbm.at[row, pl.ds(col, B_PER_W)], idx_v)

    gsem = (gsem0, gsem1)
    osem = (osem0, osem1)

    def half(parity):
        return rows_v.at[pl.ds(parity * PAIR, PAIR)]

    def pair(p, parity, waited):
        i0 = 2 * p
        if waited:
            pltpu.make_async_copy(
                half(parity),
                out_hbm.at[pl.ds(base + (p - 2) * PAIR, PAIR)],
                osem[parity]).wait()
        g0 = pltpu.async_copy(
            table_hbm.at[idx_v.at[pl.ds(i0 * CHUNK, CHUNK)]],
            rows_v.at[pl.ds(parity * PAIR, CHUNK)], gsem[0])
        g1 = pltpu.async_copy(
            table_hbm.at[idx_v.at[pl.ds((i0 + 1) * CHUNK, CHUNK)]],
            rows_v.at[pl.ds(parity * PAIR + CHUNK, CHUNK)], gsem[1])
        g0.wait()
        g1.wait()
        pltpu.async_copy(half(parity),
                         out_hbm.at[pl.ds(base + p * PAIR, PAIR)],
                         osem[parity])

    pair(0, 0, waited=False)
    pair(1, 1, waited=False)

    @pl.loop(2, NPAIR, step=2)
    def _middle(pv):
        for dp in range(2):
            pair(pv + dp, dp, waited=True)

    for parity, p in ((0, NPAIR - 2), (1, NPAIR - 1)):
        pltpu.make_async_copy(
            half(parity),
            out_hbm.at[pl.ds(base + p * PAIR, PAIR)],
            osem[parity]).wait()


@functools.partial(jax.jit, static_argnames=())
def _sc_gather(wte, ids2d):
    mesh = plsc.VectorSubcoreMesh(
        core_axis_name="c", subcore_axis_name="s",
        num_cores=NC, num_subcores=NS,
    )
    return pl.kernel(
        _gather_body,
        out_type=jax.ShapeDtypeStruct((B_FLAT, D), jnp.float32),
        mesh=mesh,
        scratch_types=[
            pltpu.VMEM((B_PER_W,), jnp.int32),
            pltpu.VMEM((2 * PAIR, D), jnp.float32),
            pltpu.SemaphoreType.DMA,
            pltpu.SemaphoreType.DMA,
            pltpu.SemaphoreType.DMA,
            pltpu.SemaphoreType.DMA,
        ],
    )(wte, ids2d)


def _mask_body(am_ref, out_ref):
    m = am_ref[...].astype(jnp.bfloat16)
    out_ref[...] = (1.0 - m) * jnp.finfo(jnp.bfloat16).min


def _mask_expand(attention_mask):
    b, s = attention_mask.shape
    return pl.pallas_call(
        _mask_body,
        out_shape=jax.ShapeDtypeStruct((b, s), jnp.bfloat16),
    )(attention_mask)


def kernel(input_ids, attention_mask, wte):
    input_shape = input_ids.shape
    hidden = _sc_gather(wte, input_ids)
    hidden = hidden.reshape(*input_shape, D)
    batch = input_ids.reshape(-1, input_shape[-1]).shape[0]
    am = _mask_expand(attention_mask.reshape(batch, -1))
    am = am[:, None, None, :]
    return (hidden, am)
